# Initial kernel scaffold; baseline (speedup 1.0000x reference)
import jax
import jax.numpy as jnp
from jax.experimental import pallas as pl


def kernel(tokens, table):
    raise NotImplementedError("write your pallas kernel here")


# speedup vs baseline: 2.4159x; 2.4159x over previous
import functools
import math

import jax
import jax.numpy as jnp
from jax import lax
from jax.experimental import pallas as pl
from jax.experimental.pallas import tpu as pltpu
from jax.experimental.pallas import tpu_sc as plsc

N_EMBD = 128
SCALE = math.sqrt(N_EMBD)

NC, NS, L = 2, 16, 16
NW = NC * NS

B = 4096 * 50
B_PER_W = B // NW
CHUNK = 128
NCHUNK = B_PER_W // CHUNK


@jax.jit
def _sc_embed(tokens_flat, table):
    mesh = plsc.VectorSubcoreMesh(core_axis_name="c", subcore_axis_name="s")

    @functools.partial(
        pl.kernel,
        out_type=jax.ShapeDtypeStruct((B, N_EMBD), jnp.float32),
        mesh=mesh,
        scratch_types=[
            pltpu.VMEM((NCHUNK, CHUNK), jnp.int32),
            pltpu.VMEM((CHUNK, N_EMBD), jnp.float32),
            pltpu.SemaphoreType.DMA,
        ],
    )
    def k(idx_hbm, table_hbm, out_hbm, idx_v, rows_v, sem):
        wid = lax.axis_index("s") * NC + lax.axis_index("c")
        base = wid * B_PER_W
        pltpu.sync_copy(idx_hbm.at[wid], idx_v)

        def chunk_body(j, carry):
            pltpu.async_copy(table_hbm.at[idx_v.at[j]], rows_v, sem).wait()

            def row_body(r, c2):
                for c in range(N_EMBD // L):
                    sl = pl.ds(c * L, L)
                    rows_v[r, sl] = rows_v[r, sl] * SCALE
                return c2

            lax.fori_loop(0, CHUNK, row_body, 0)
            pltpu.sync_copy(rows_v, out_hbm.at[pl.ds(base + j * CHUNK, CHUNK)])
            return carry

        lax.fori_loop(0, NCHUNK, chunk_body, 0)

    return k(tokens_flat, table)


def kernel(tokens, table):
    tokens_flat = tokens.astype(jnp.int32).reshape(NW, NCHUNK, CHUNK)
    out = _sc_embed(tokens_flat, table)
    return out.reshape(tokens.shape[0], tokens.shape[1], N_EMBD)


# speedup vs baseline: 2.9227x; 1.2098x over previous
import functools
import math

import jax
import jax.numpy as jnp
from jax import lax
from jax.experimental import pallas as pl
from jax.experimental.pallas import tpu as pltpu
from jax.experimental.pallas import tpu_sc as plsc

N_EMBD = 128
SCALE = math.sqrt(N_EMBD)

NC, NS, L = 2, 16, 16
NW = NC * NS

B = 4096 * 50
B_PER_W = B // NW
CHUNK = 128
NCHUNK = B_PER_W // CHUNK
NBUF = 4
LOOKAHEAD = 2


@jax.jit
def _sc_embed(tokens_flat, table):
    mesh = plsc.VectorSubcoreMesh(core_axis_name="c", subcore_axis_name="s")

    @functools.partial(
        pl.kernel,
        out_type=jax.ShapeDtypeStruct((B, N_EMBD), jnp.float32),
        mesh=mesh,
        scratch_types=[
            pltpu.VMEM((NCHUNK, CHUNK), jnp.int32),
            [pltpu.VMEM((CHUNK, N_EMBD), jnp.float32) for _ in range(NBUF)],
            [pltpu.SemaphoreType.DMA for _ in range(NBUF)],
            [pltpu.SemaphoreType.DMA for _ in range(NBUF)],
        ],
    )
    def k(idx_hbm, table_hbm, out_hbm, idx_v, bufs, gsems, ssems):
        wid = lax.axis_index("s") * NC + lax.axis_index("c")
        base = wid * B_PER_W
        pltpu.sync_copy(idx_hbm.at[wid], idx_v)

        def start_gather(j):
            b = j % NBUF
            return pltpu.async_copy(table_hbm.at[idx_v.at[j]], bufs[b], gsems[b])

        def start_store(j):
            b = j % NBUF
            return pltpu.async_copy(
                bufs[b], out_hbm.at[pl.ds(base + j * CHUNK, CHUNK)], ssems[b])

        gathers = {}
        stores = {}
        for j in range(LOOKAHEAD):
            gathers[j] = start_gather(j)

        for j in range(NCHUNK):
            jk = j + LOOKAHEAD
            if jk < NCHUNK:
                if jk - NBUF >= 0:
                    stores.pop(jk - NBUF).wait()
                gathers[jk] = start_gather(jk)
            gathers.pop(j).wait()

            buf = bufs[j % NBUF]

            def row_body(r, carry, buf=buf):
                for c in range(N_EMBD // L):
                    sl = pl.ds(c * L, L)
                    buf[r, sl] = buf[r, sl] * SCALE
                return carry

            lax.fori_loop(0, CHUNK, row_body, 0)
            stores[j] = start_store(j)

        for j in sorted(stores):
            stores.pop(j).wait()

    return k(tokens_flat, table)


def kernel(tokens, table):
    tokens_flat = tokens.astype(jnp.int32).reshape(NW, NCHUNK, CHUNK)
    out = _sc_embed(tokens_flat, table)
    return out.reshape(tokens.shape[0], tokens.shape[1], N_EMBD)


# speedup vs baseline: 5.2461x; 1.7949x over previous
import functools
import math

import jax
import jax.numpy as jnp
from jax import lax
from jax.experimental import pallas as pl
from jax.experimental.pallas import tpu as pltpu
from jax.experimental.pallas import tpu_sc as plsc

N_EMBD = 128
SCALE = math.sqrt(N_EMBD)

NC, NS, L = 2, 16, 16
NW = NC * NS

NSENTS = 4096
SLEN = 50
S_PER_W = NSENTS // NW
NBUF = 8
LA = 4


@jax.jit
def _sc_embed(tokens_g, table):
    mesh = plsc.VectorSubcoreMesh(core_axis_name="c", subcore_axis_name="s")

    @functools.partial(
        pl.kernel,
        out_type=jax.ShapeDtypeStruct((NSENTS, SLEN, N_EMBD), jnp.float32),
        mesh=mesh,
        scratch_types=[
            pltpu.VMEM((S_PER_W, SLEN), jnp.int32),
            [pltpu.VMEM((SLEN, N_EMBD), jnp.float32) for _ in range(NBUF)],
            [pltpu.SemaphoreType.DMA for _ in range(NBUF)],
            [pltpu.SemaphoreType.DMA for _ in range(NBUF)],
        ],
    )
    def k(idx_hbm, table_hbm, out_hbm, idx_v, bufs, gsems, ssems):
        wid = lax.axis_index("s") * NC + lax.axis_index("c")
        sent0 = wid * S_PER_W
        pltpu.sync_copy(idx_hbm.at[wid], idx_v)

        def gather(s, b):
            pltpu.async_copy(table_hbm.at[idx_v.at[s]], bufs[b], gsems[b])

        def store(s, b):
            pltpu.async_copy(bufs[b], out_hbm.at[sent0 + s], ssems[b])

        def wait_gather(b):
            pltpu.make_async_copy(
                table_hbm.at[idx_v.at[0]], bufs[b], gsems[b]).wait()

        def wait_store(b):
            pltpu.make_async_copy(
                bufs[b], out_hbm.at[sent0], ssems[b]).wait()

        def scale(b):
            def row_body(r, carry):
                for c in range(N_EMBD // L):
                    sl = pl.ds(c * L, L)
                    bufs[b][r, sl] = bufs[b][r, sl] * SCALE
                return carry

            lax.fori_loop(0, SLEN, row_body, 0)

        def consume(s, b, b_pre, prefetch, reuse):
            if prefetch:
                if reuse:
                    wait_store(b_pre)
                gather(s + LA, b_pre)
            wait_gather(b)
            scale(b)
            store(s, b)

        for s in range(LA):
            gather(s, s % NBUF)
        for s in range(LA):
            consume(s, s % NBUF, (s + LA) % NBUF, True, False)

        @pl.loop(LA, S_PER_W - LA, step=NBUF)
        def body(g):
            for b_off in range(NBUF):
                b = (LA + b_off) % NBUF
                consume(g + b_off, b, (b + LA) % NBUF, True, True)

        for s in range(S_PER_W - LA, S_PER_W):
            consume(s, s % NBUF, 0, False, False)

        for b in range(NBUF):
            wait_store(b)

    return k(tokens_g, table)


def kernel(tokens, table):
    tokens_g = tokens.astype(jnp.int32).reshape(NW, S_PER_W, SLEN)
    return _sc_embed(tokens_g, table)


# speedup vs baseline: 5.2511x; 1.0009x over previous
import functools
import math

import jax
import jax.numpy as jnp
from jax import lax
from jax.experimental import pallas as pl
from jax.experimental.pallas import tpu as pltpu
from jax.experimental.pallas import tpu_sc as plsc

N_EMBD = 128
SCALE = math.sqrt(N_EMBD)

NC, NS, L = 2, 16, 16
NW = NC * NS

NSENTS = 4096
SLEN = 50
S_PER_W = NSENTS // NW
NBUF = 8
LA = 4


@jax.jit
def _sc_embed(tokens_g, table):
    mesh = plsc.VectorSubcoreMesh(core_axis_name="c", subcore_axis_name="s")

    @functools.partial(
        pl.kernel,
        out_type=jax.ShapeDtypeStruct((NSENTS, SLEN, N_EMBD), jnp.float32),
        mesh=mesh,
        compiler_params=pltpu.CompilerParams(use_tc_tiling_on_sc=True),
        scratch_types=[
            pltpu.VMEM((S_PER_W, SLEN), jnp.int32),
            [pltpu.VMEM((SLEN, N_EMBD), jnp.float32) for _ in range(NBUF)],
            [pltpu.SemaphoreType.DMA for _ in range(NBUF)],
            [pltpu.SemaphoreType.DMA for _ in range(NBUF)],
        ],
    )
    def k(idx_hbm, table_hbm, out_hbm, idx_v, bufs, gsems, ssems):
        wid = lax.axis_index("s") * NC + lax.axis_index("c")
        sent0 = wid * S_PER_W
        pltpu.sync_copy(idx_hbm.at[wid], idx_v)

        def gather(s, b):
            pltpu.async_copy(table_hbm.at[idx_v.at[s]], bufs[b], gsems[b])

        def store(s, b):
            pltpu.async_copy(bufs[b], out_hbm.at[sent0 + s], ssems[b])

        def wait_gather(b):
            pltpu.make_async_copy(
                table_hbm.at[idx_v.at[0]], bufs[b], gsems[b]).wait()

        def wait_store(b):
            pltpu.make_async_copy(
                bufs[b], out_hbm.at[sent0], ssems[b]).wait()

        def scale(b):
            def row_body(r, carry):
                for c in range(N_EMBD // L):
                    sl = pl.ds(c * L, L)
                    bufs[b][r, sl] = bufs[b][r, sl] * SCALE
                return carry

            lax.fori_loop(0, SLEN, row_body, 0)

        def consume(s, b, b_pre, prefetch, reuse):
            if prefetch:
                if reuse:
                    wait_store(b_pre)
                gather(s + LA, b_pre)
            wait_gather(b)
            scale(b)
            store(s, b)

        for s in range(LA):
            gather(s, s % NBUF)
        for s in range(LA):
            consume(s, s % NBUF, (s + LA) % NBUF, True, False)

        @pl.loop(LA, S_PER_W - LA, step=NBUF)
        def body(g):
            for b_off in range(NBUF):
                b = (LA + b_off) % NBUF
                consume(g + b_off, b, (b + LA) % NBUF, True, True)

        for s in range(S_PER_W - LA, S_PER_W):
            consume(s, s % NBUF, 0, False, False)

        for b in range(NBUF):
            wait_store(b)

    return k(tokens_g, table)


def kernel(tokens, table):
    tokens_g = tokens.astype(jnp.int32).reshape(NW, S_PER_W, SLEN)
    return _sc_embed(tokens_g, table)


# speedup vs baseline: 9.1566x; 1.7438x over previous
import functools
import math

import jax
import jax.numpy as jnp
from jax import lax
from jax.experimental import pallas as pl
from jax.experimental.pallas import tpu as pltpu
from jax.experimental.pallas import tpu_sc as plsc

N_EMBD = 128
SCALE = math.sqrt(N_EMBD)

NC, NS, L = 2, 16, 16
NW = NC * NS

NSENTS = 4096
SLEN = 50
S_PER_W = NSENTS // NW
NBUF = 6
LA = 3


@jax.jit
def _sc_embed(tokens_g, table):
    mesh = plsc.VectorSubcoreMesh(core_axis_name="c", subcore_axis_name="s")

    @functools.partial(
        pl.kernel,
        out_type=jax.ShapeDtypeStruct((SLEN, NSENTS, N_EMBD), jnp.float32),
        mesh=mesh,
        scratch_types=[
            pltpu.VMEM((SLEN, S_PER_W), jnp.int32),
            [pltpu.VMEM((S_PER_W, N_EMBD), jnp.float32) for _ in range(NBUF)],
            [pltpu.SemaphoreType.DMA for _ in range(NBUF)],
            [pltpu.SemaphoreType.DMA for _ in range(NBUF)],
        ],
    )
    def k(idx_hbm, table_hbm, out_hbm, idx_v, bufs, gsems, ssems):
        wid = lax.axis_index("s") * NC + lax.axis_index("c")
        i0 = wid * S_PER_W
        pltpu.sync_copy(idx_hbm.at[wid], idx_v)

        def gather(j):
            b = j % NBUF
            return pltpu.async_copy(
                table_hbm.at[idx_v.at[j]], bufs[b], gsems[b])

        def store(j):
            b = j % NBUF
            return pltpu.async_copy(
                bufs[b], out_hbm.at[j, pl.ds(i0, S_PER_W)], ssems[b])

        def scale(j):
            buf = bufs[j % NBUF]

            def row_body(r, carry):
                for c in range(N_EMBD // L):
                    sl = pl.ds(c * L, L)
                    buf[r, sl] = buf[r, sl] * SCALE
                return carry

            lax.fori_loop(0, S_PER_W, row_body, 0)

        gathers, stores = {}, {}
        for j in range(LA):
            gathers[j] = gather(j)

        for j in range(SLEN):
            t = j + LA
            if t < SLEN:
                if t - NBUF >= 0:
                    stores.pop(t - NBUF).wait()
                gathers[t] = gather(t)
            gathers.pop(j).wait()
            scale(j)
            stores[j] = store(j)

        for j in sorted(stores):
            stores.pop(j).wait()

    return k(tokens_g, table)


def kernel(tokens, table):
    tokens_g = tokens.astype(jnp.int32).reshape(NW, S_PER_W, SLEN)
    tokens_g = tokens_g.transpose(0, 2, 1)
    out = _sc_embed(tokens_g, table)
    return out.transpose(1, 0, 2)
